# Initial kernel scaffold; baseline (speedup 1.0000x reference)
import jax
import jax.numpy as jnp
from jax.experimental import pallas as pl


def kernel(token_ids, embedding):
    raise NotImplementedError("write your pallas kernel here")


# speedup vs baseline: 1.8628x; 1.8628x over previous
import functools

import jax
import jax.numpy as jnp
from jax import lax
from jax.experimental import pallas as pl
from jax.experimental.pallas import tpu as pltpu
from jax.experimental.pallas import tpu_sc as plsc


def _build_gather(B, D, C, NC, NW, b_per_w):
    n_chunks = b_per_w // C
    mesh = plsc.VectorSubcoreMesh(core_axis_name="c", subcore_axis_name="s")

    @functools.partial(
        pl.kernel,
        mesh=mesh,
        out_type=jax.ShapeDtypeStruct((B, D), jnp.float32),
        scratch_types=[
            pltpu.VMEM((C,), jnp.int32),
            pltpu.VMEM((C, D), jnp.float32),
            pltpu.SemaphoreType.DMA,
        ],
        compiler_params=pltpu.CompilerParams(use_tc_tiling_on_sc=False),
    )
    def gather_kernel(ids_hbm, table_hbm, out_hbm, idx_v, rows_v, sem):
        wid = lax.axis_index("s") * NC + lax.axis_index("c")
        base = pl.multiple_of(wid * b_per_w, 8)

        def body(i, carry):
            off = pl.multiple_of(base + i * C, 8)
            pltpu.sync_copy(ids_hbm.at[pl.ds(off, C)], idx_v)
            pltpu.async_copy(table_hbm.at[idx_v], rows_v, sem).wait()
            pltpu.sync_copy(rows_v, out_hbm.at[pl.ds(off, C)])
            return carry

        lax.fori_loop(0, n_chunks, body, 0)

    return gather_kernel


def kernel(token_ids, embedding):
    B0, S = token_ids.shape
    D = embedding.shape[1]
    B = B0 * S
    flat_ids = token_ids.reshape(B).astype(jnp.int32)

    info = plsc.get_sparse_core_info()
    NC, NS = info.num_cores, info.num_subcores
    NW = NC * NS
    b_per_w = B // NW
    C = 1600

    out = _build_gather(B, D, C, NC, NW, b_per_w)(flat_ids, embedding)
    return out.reshape(B0, S, D)


# speedup vs baseline: 1.8757x; 1.0069x over previous
import functools

import jax
import jax.numpy as jnp
from jax import lax
from jax.experimental import pallas as pl
from jax.experimental.pallas import tpu as pltpu
from jax.experimental.pallas import tpu_sc as plsc


def _build_gather(B, D, C, NC, NW, b_per_w):
    n_chunks = b_per_w // C
    n_pairs = n_chunks // 2
    mesh = plsc.VectorSubcoreMesh(core_axis_name="c", subcore_axis_name="s")

    @functools.partial(
        pl.kernel,
        mesh=mesh,
        out_type=jax.ShapeDtypeStruct((B, D), jnp.float32),
        scratch_types=[
            pltpu.VMEM((b_per_w,), jnp.int32),
            pltpu.VMEM((2, C, D), jnp.float32),
            pltpu.SemaphoreType.DMA,
            pltpu.SemaphoreType.DMA,
        ],
        compiler_params=pltpu.CompilerParams(use_tc_tiling_on_sc=False),
    )
    def gather_kernel(ids_hbm, table_hbm, out_hbm, idx_v, rows_v, sem0, sem1):
        sems = (sem0, sem1)
        wid = lax.axis_index("s") * NC + lax.axis_index("c")
        base = pl.multiple_of(wid * b_per_w, 8)

        pltpu.sync_copy(ids_hbm.at[pl.ds(base, b_per_w)], idx_v)

        def start(i, b):
            pltpu.async_copy(
                table_hbm.at[idx_v.at[pl.ds(i * C, C)]], rows_v.at[b], sems[b]
            )

        def finish(i, b):
            pltpu.make_async_copy(
                table_hbm.at[idx_v.at[pl.ds(i * C, C)]], rows_v.at[b], sems[b]
            ).wait()
            off = pl.multiple_of(base + i * C, 8)
            pltpu.sync_copy(rows_v.at[b], out_hbm.at[pl.ds(off, C)])

        start(0, 0)
        start(1, 1)

        def body(j, carry):
            for b in range(2):
                i = j * 2 + b
                finish(i - 2, b)
                start(i, b)
            return carry

        lax.fori_loop(1, n_pairs, body, 0)
        finish(n_chunks - 2, 0)
        finish(n_chunks - 1, 1)

    return gather_kernel


def kernel(token_ids, embedding):
    B0, S = token_ids.shape
    D = embedding.shape[1]
    B = B0 * S
    flat_ids = token_ids.reshape(B).astype(jnp.int32)

    info = plsc.get_sparse_core_info()
    NC, NS = info.num_cores, info.num_subcores
    NW = NC * NS
    b_per_w = B // NW
    C = 800

    out = _build_gather(B, D, C, NC, NW, b_per_w)(flat_ids, embedding)
    return out.reshape(B0, S, D)


# speedup vs baseline: 1.9562x; 1.0429x over previous
import functools

import jax
import jax.numpy as jnp
from jax import lax
from jax.experimental import pallas as pl
from jax.experimental.pallas import tpu as pltpu
from jax.experimental.pallas import tpu_sc as plsc


def _build_gather(B, D, C, NC, NW, b_per_w):
    n_chunks = b_per_w // C
    n_pairs = n_chunks // 2
    mesh = plsc.VectorSubcoreMesh(core_axis_name="c", subcore_axis_name="s")

    @functools.partial(
        pl.kernel,
        mesh=mesh,
        out_type=jax.ShapeDtypeStruct((B, D), jnp.float32),
        scratch_types=[
            pltpu.VMEM((b_per_w,), jnp.int32),
            pltpu.VMEM((2, C, D), jnp.float32),
            pltpu.SemaphoreType.DMA,
            pltpu.SemaphoreType.DMA,
        ],
        compiler_params=pltpu.CompilerParams(use_tc_tiling_on_sc=False),
    )
    def gather_kernel(ids_hbm, table_hbm, out_hbm, idx_v, rows_v, sem0, sem1):
        sems = (sem0, sem1)
        wid = lax.axis_index("s") * NC + lax.axis_index("c")
        base = pl.multiple_of(wid * b_per_w, 8)

        pltpu.sync_copy(ids_hbm.at[pl.ds(base, b_per_w)], idx_v)

        def start(i, b):
            pltpu.async_copy(
                table_hbm.at[idx_v.at[pl.ds(i * C, C)]], rows_v.at[b], sems[b]
            )

        def finish(i, b):
            pltpu.make_async_copy(
                table_hbm.at[idx_v.at[pl.ds(i * C, C)]], rows_v.at[b], sems[b]
            ).wait()
            off = pl.multiple_of(base + i * C, 8)
            pltpu.sync_copy(rows_v.at[b], out_hbm.at[pl.ds(off, C)])

        start(0, 0)
        start(1, 1)

        def body(j, carry):
            for b in range(2):
                i = j * 2 + b
                finish(i - 2, b)
                start(i, b)
            return carry

        lax.fori_loop(1, n_pairs, body, 0)
        finish(n_chunks - 2, 0)
        finish(n_chunks - 1, 1)

    return gather_kernel


def kernel(token_ids, embedding):
    B0, S = token_ids.shape
    D = embedding.shape[1]
    B = B0 * S
    flat_ids = token_ids.T.reshape(B).astype(jnp.int32)

    info = plsc.get_sparse_core_info()
    NC, NS = info.num_cores, info.num_subcores
    NW = NC * NS
    b_per_w = B // NW
    C = 800

    out = _build_gather(B, D, C, NC, NW, b_per_w)(flat_ids, embedding)
    return out.reshape(S, B0, D).transpose(1, 0, 2)
